# baseline (speedup 1.0000x reference)
import functools
import jax
import jax.numpy as jnp
from jax import lax
from jax.experimental import pallas as pl
from jax.experimental.pallas import tpu as pltpu
from jax.experimental.pallas import tpu_sc as plsc

R, C, KTOP = 128, 32768, 64
NC, NS, L = 2, 16, 16
NW = NC * NS
RPW = R // NW
NV = C // L
NB = 256
NBS = 257
CAP = 4096
UH = 8
UM = 16
UCP = 4
IMIN = -0x80000000


def _skey(v):
    b = plsc.bitcast(v, jnp.int32)
    m = lax.shift_right_arithmetic(b, 31)
    return b ^ (m & jnp.int32(0x7FFFFFFF))


def _body(x_hbm, o_hbm, rowa, rowb, rowc, cbuf, hist, sbuf,
          si0, si1, si2, si3, so0, so1, so2, so3):
    wid = lax.axis_index("s") * NC + lax.axis_index("c")
    base = wid * RPW
    lanes = lax.iota(jnp.int32, L)
    lane_base = lanes * NBS
    ones = jnp.ones((L,), jnp.int32)
    zeros = jnp.zeros((L,), jnp.int32)

    bufs = [rowa, rowb, rowc, rowa]
    isems = [si0, si1, si2, si3]
    osems = [so0, so1, so2, so3]

    @plsc.parallel_loop(0, (L * NBS + L - 1) // L, unroll=8)
    def _(i):
        hist[pl.ds(i * L, L)] = zeros

    in_h = [pltpu.async_copy(x_hbm.at[base + r], bufs[r], isems[r])
            for r in range(3)]
    out_h = [None] * RPW

    for r in range(RPW):
        buf = bufs[r]
        with jax.named_scope("p_wait_in"):
            in_h[r].wait()

        with jax.named_scope("p_hist"):
            @plsc.parallel_loop(0, NV, unroll=UH)
            def _(i):
                v = buf[pl.ds(i * L, L)]
                d = lax.shift_right_arithmetic(_skey(v), 24) + 128
                plsc.addupdate_scatter(hist, [lane_base + d], ones)

        sbuf[pl.ds(NB, L)] = zeros

        def sj(t, carry_cge):
            carry, cge = carry_cge
            j = (NB // L) - 1 - t
            acc = zeros
            for l in range(L):
                acc = acc + hist[pl.ds(l * NBS + j * L, L)]
                hist[pl.ds(l * NBS + j * L, L)] = zeros
            s = lax.rev(jnp.cumsum(lax.rev(acc, (0,)), axis=0), (0,)) + carry
            sbuf[pl.ds(j * L, L)] = s
            cge = cge + jnp.max(plsc.all_reduce_population_count(s >= KTOP))
            return jnp.max(s), cge
        with jax.named_scope("p_scan"):
            _, cge = lax.fori_loop(0, NB // L, sj,
                                   (jnp.int32(0), jnp.int32(0)))
        d0 = cge - 1
        snext = plsc.load_gather(sbuf, [jnp.broadcast_to(d0 + 1, (L,))])
        krem = KTOP - jnp.max(snext)

        def _decode(kv):
            kb = jnp.broadcast_to(kv, (L,))
            kb = jnp.where(kb < 0, kb ^ jnp.int32(0x7FFFFFFF), kb)
            return plsc.bitcast(kb, jnp.float32)
        lo_f = _decode(lax.shift_left(d0 - 128, 24))
        hi_f = jnp.where(d0 == 255, jnp.float32(jnp.inf),
                         _decode(lax.shift_left(d0 - 127, 24)))

        with jax.named_scope("p_compress"):
            @plsc.parallel_loop(0, NV, unroll=UCP, carry=zeros - 1)
            def offv(i, off):
                v = buf[pl.ds(i * L, L)]
                msk = (v >= lo_f) & (v < hi_f)
                addr = jnp.minimum(off + plsc.cumsum(ones, mask=msk),
                                   CAP - 1)
                plsc.store_scatter(cbuf, [addr], plsc.bitcast(v, jnp.int32),
                                   mask=msk)
                return off + plsc.all_reduce_population_count(msk)
        cnt = jnp.max(offv) + 1
        pbase = jnp.minimum(cnt, CAP)
        cbuf[pl.ds(pbase, L)] = jnp.broadcast_to(jnp.int32(IMIN), (L,))
        cbuf[pl.ds(pbase + L, L)] = jnp.broadcast_to(jnp.int32(IMIN), (L,))

        over = (cnt > CAP) | (d0 <= 128)
        prefix0 = jnp.broadcast_to(lax.shift_left(d0 - 128, 24), (L,))

        def cbuf_search(_):
            nv_c = (cnt + (2 * L - 1)) // (2 * L)
            kremv = jnp.broadcast_to(krem, (L,))

            def bit_step(t, prefix):
                cand = prefix | lax.shift_left(jnp.int32(1), 23 - t)

                @plsc.parallel_loop(0, nv_c, carry=(zeros, zeros))
                def accs(i, acc):
                    a0, a1 = acc
                    s0 = cbuf[pl.ds(i * 2 * L, L)]
                    s1 = cbuf[pl.ds((i * 2 + 1) * L, L)]
                    return (
                        a0 + plsc.all_reduce_population_count(s0 >= cand),
                        a1 + plsc.all_reduce_population_count(s1 >= cand))
                return jnp.where(accs[0] + accs[1] >= kremv, cand, prefix)
            return lax.fori_loop(0, 24, bit_step, prefix0)

        def row_search(_):
            kv = jnp.broadcast_to(jnp.int32(KTOP), (L,))

            def bit_step(t, prefix):
                cand = prefix | lax.shift_left(jnp.int32(1), 23 - t)

                @plsc.parallel_loop(0, NV, unroll=4, carry=zeros)
                def nvec(i, acc):
                    sk = _skey(buf[pl.ds(i * L, L)])
                    return acc + plsc.all_reduce_population_count(sk >= cand)
                return jnp.where(nvec >= kv, cand, prefix)
            return lax.fori_loop(0, 24, bit_step, prefix0)

        with jax.named_scope("p_search"):
            tkey = lax.cond(over, row_search, cbuf_search, 0)

        tb = jnp.where(tkey < 0, tkey ^ jnp.int32(0x7FFFFFFF), tkey)
        tf = plsc.bitcast(tb, jnp.float32)

        with jax.named_scope("p_mask"):
            @plsc.parallel_loop(0, NV, unroll=UM)
            def _(i):
                v = buf[pl.ds(i * L, L)]
                buf[pl.ds(i * L, L)] = jnp.where(v >= tf, v, jnp.float32(0))

        out_h[r] = pltpu.async_copy(buf, o_hbm.at[base + r], osems[r])
        if r == 1:
            out_h[0].wait()
            in_h.append(pltpu.async_copy(x_hbm.at[base + 3],
                                         bufs[3], isems[3]))

    for r in range(1, RPW):
        out_h[r].wait()


@jax.jit
def kernel(x):
    mesh = plsc.VectorSubcoreMesh(core_axis_name="c", subcore_axis_name="s",
                                  num_cores=NC, num_subcores=NS)
    run = pl.kernel(
        _body,
        out_type=jax.ShapeDtypeStruct((R, C), jnp.float32),
        mesh=mesh,
        compiler_params=pltpu.CompilerParams(needs_layout_passes=False),
        scratch_types=[
            pltpu.VMEM((C,), jnp.float32),
            pltpu.VMEM((C,), jnp.float32),
            pltpu.VMEM((C,), jnp.float32),
            pltpu.VMEM((CAP + 2 * L,), jnp.int32),
            pltpu.VMEM((L * NBS + L,), jnp.int32),
            pltpu.VMEM((NB + L,), jnp.int32),
            pltpu.SemaphoreType.DMA,
            pltpu.SemaphoreType.DMA,
            pltpu.SemaphoreType.DMA,
            pltpu.SemaphoreType.DMA,
            pltpu.SemaphoreType.DMA,
            pltpu.SemaphoreType.DMA,
            pltpu.SemaphoreType.DMA,
            pltpu.SemaphoreType.DMA,
        ],
    )
    return run(x)


# speedup vs baseline: 1.0281x; 1.0281x over previous
import functools
import jax
import jax.numpy as jnp
from jax import lax
from jax.experimental import pallas as pl
from jax.experimental.pallas import tpu as pltpu
from jax.experimental.pallas import tpu_sc as plsc

R, C, KTOP = 128, 32768, 64
NC, NS, L = 2, 16, 16
NW = NC * NS
RPW = R // NW
NV = C // L
NB = 256
NBS = 257
HCAP = 2048
UH = 8
UM = 16
UCP = 4
IMIN = -0x80000000


def _skey(v):
    b = plsc.bitcast(v, jnp.int32)
    m = lax.shift_right_arithmetic(b, 31)
    return b ^ (m & jnp.int32(0x7FFFFFFF))


def _body(x_hbm, o_hbm, rowa, rowb, rowc, cbuf, hist, sbuf,
          si0, si1, si2, si3, so0, so1, so2, so3):
    wid = lax.axis_index("s") * NC + lax.axis_index("c")
    base = wid * RPW
    lanes = lax.iota(jnp.int32, L)
    lane_base = lanes * NBS
    ones = jnp.ones((L,), jnp.int32)
    zeros = jnp.zeros((L,), jnp.int32)

    bufs = [rowa, rowb, rowc, rowa]
    isems = [si0, si1, si2, si3]
    osems = [so0, so1, so2, so3]

    @plsc.parallel_loop(0, (L * NBS + L - 1) // L, unroll=8)
    def _(i):
        hist[pl.ds(i * L, L)] = zeros

    in_h = [pltpu.async_copy(x_hbm.at[base + r], bufs[r], isems[r])
            for r in range(3)]
    out_h = [None] * RPW

    for r in range(RPW):
        buf = bufs[r]
        in_h[r].wait()

        @plsc.parallel_loop(0, NV, unroll=UH)
        def _(i):
            v = buf[pl.ds(i * L, L)]
            d = lax.shift_right_arithmetic(_skey(v), 24) + 128
            plsc.addupdate_scatter(hist, [lane_base + d], ones)

        sbuf[pl.ds(NB, L)] = zeros

        def sj(t, carry_cge):
            carry, cge = carry_cge
            j = (NB // L) - 1 - t
            acc = zeros
            for l in range(L):
                acc = acc + hist[pl.ds(l * NBS + j * L, L)]
                hist[pl.ds(l * NBS + j * L, L)] = zeros
            s = lax.rev(jnp.cumsum(lax.rev(acc, (0,)), axis=0), (0,)) + carry
            sbuf[pl.ds(j * L, L)] = s
            cge = cge + jnp.max(plsc.all_reduce_population_count(s >= KTOP))
            return jnp.max(s), cge
        _, cge = lax.fori_loop(0, NB // L, sj, (jnp.int32(0), jnp.int32(0)))
        d0 = cge - 1
        snext = plsc.load_gather(sbuf, [jnp.broadcast_to(d0 + 1, (L,))])
        krem = KTOP - jnp.max(snext)

        def _decode(kv):
            kb = jnp.broadcast_to(kv, (L,))
            kb = jnp.where(kb < 0, kb ^ jnp.int32(0x7FFFFFFF), kb)
            return plsc.bitcast(kb, jnp.float32)
        lo_f = _decode(lax.shift_left(d0 - 128, 24))
        hi_f = jnp.where(d0 == 255, jnp.float32(jnp.inf),
                         _decode(lax.shift_left(d0 - 127, 24)))

        iminv = jnp.broadcast_to(jnp.int32(IMIN), (L,))

        @plsc.parallel_loop(0, (2 * HCAP) // L, unroll=8)
        def _(i):
            cbuf[pl.ds(i * L, L)] = iminv

        @plsc.parallel_loop(0, NV // 2, unroll=UCP, carry=(zeros - 1,
                                                           zeros - 1))
        def offv(i, offab):
            offa, offb = offab
            va = buf[pl.ds((2 * i) * L, L)]
            vb = buf[pl.ds((2 * i + 1) * L, L)]
            ma = (va >= lo_f) & (va < hi_f)
            mb = (vb >= lo_f) & (vb < hi_f)
            aa = jnp.minimum(offa + plsc.cumsum(ones, mask=ma), HCAP - 1)
            ab = jnp.minimum(offb + plsc.cumsum(ones, mask=mb),
                             HCAP - 1) + HCAP
            plsc.store_scatter(cbuf, [aa], plsc.bitcast(va, jnp.int32),
                               mask=ma)
            plsc.store_scatter(cbuf, [ab], plsc.bitcast(vb, jnp.int32),
                               mask=mb)
            return (offa + plsc.all_reduce_population_count(ma),
                    offb + plsc.all_reduce_population_count(mb))
        cnta = jnp.max(offv[0]) + 1
        cntb = jnp.max(offv[1]) + 1

        over = (cnta > HCAP) | (cntb > HCAP) | (d0 <= 128)
        prefix0 = jnp.broadcast_to(lax.shift_left(d0 - 128, 24), (L,))

        def cbuf_search(_):
            nv_c = (jnp.maximum(cnta, cntb) + (L - 1)) // L
            kremv = jnp.broadcast_to(krem, (L,))

            def bit_step(t, prefix):
                cand = prefix | lax.shift_left(jnp.int32(1), 23 - t)

                @plsc.parallel_loop(0, nv_c, carry=(zeros, zeros))
                def accs(i, acc):
                    a0, a1 = acc
                    s0 = cbuf[pl.ds(i * L, L)]
                    s1 = cbuf[pl.ds(HCAP + i * L, L)]
                    return (
                        a0 + plsc.all_reduce_population_count(s0 >= cand),
                        a1 + plsc.all_reduce_population_count(s1 >= cand))
                return jnp.where(accs[0] + accs[1] >= kremv, cand, prefix)
            return lax.fori_loop(0, 24, bit_step, prefix0)

        def row_search(_):
            kv = jnp.broadcast_to(jnp.int32(KTOP), (L,))

            def bit_step(t, prefix):
                cand = prefix | lax.shift_left(jnp.int32(1), 23 - t)

                @plsc.parallel_loop(0, NV, unroll=4, carry=zeros)
                def nvec(i, acc):
                    sk = _skey(buf[pl.ds(i * L, L)])
                    return acc + plsc.all_reduce_population_count(sk >= cand)
                return jnp.where(nvec >= kv, cand, prefix)
            return lax.fori_loop(0, 24, bit_step, prefix0)

        tkey = lax.cond(over, row_search, cbuf_search, 0)

        tb = jnp.where(tkey < 0, tkey ^ jnp.int32(0x7FFFFFFF), tkey)
        tf = plsc.bitcast(tb, jnp.float32)

        @plsc.parallel_loop(0, NV, unroll=UM)
        def _(i):
            v = buf[pl.ds(i * L, L)]
            buf[pl.ds(i * L, L)] = jnp.where(v >= tf, v, jnp.float32(0))

        out_h[r] = pltpu.async_copy(buf, o_hbm.at[base + r], osems[r])
        if r == 1:
            out_h[0].wait()
            in_h.append(pltpu.async_copy(x_hbm.at[base + 3],
                                         bufs[3], isems[3]))

    for r in range(1, RPW):
        out_h[r].wait()


@jax.jit
def kernel(x):
    mesh = plsc.VectorSubcoreMesh(core_axis_name="c", subcore_axis_name="s",
                                  num_cores=NC, num_subcores=NS)
    run = pl.kernel(
        _body,
        out_type=jax.ShapeDtypeStruct((R, C), jnp.float32),
        mesh=mesh,
        compiler_params=pltpu.CompilerParams(needs_layout_passes=False),
        scratch_types=[
            pltpu.VMEM((C,), jnp.float32),
            pltpu.VMEM((C,), jnp.float32),
            pltpu.VMEM((C,), jnp.float32),
            pltpu.VMEM((2 * HCAP,), jnp.int32),
            pltpu.VMEM((L * NBS + L,), jnp.int32),
            pltpu.VMEM((NB + L,), jnp.int32),
            pltpu.SemaphoreType.DMA,
            pltpu.SemaphoreType.DMA,
            pltpu.SemaphoreType.DMA,
            pltpu.SemaphoreType.DMA,
            pltpu.SemaphoreType.DMA,
            pltpu.SemaphoreType.DMA,
            pltpu.SemaphoreType.DMA,
            pltpu.SemaphoreType.DMA,
        ],
    )
    return run(x)
